# baseline (speedup 1.0000x reference)
import functools

import jax
import jax.numpy as jnp
from jax import lax
from jax.experimental import pallas as pl
from jax.experimental.pallas import tpu as pltpu
from jax.experimental.pallas import tpu_sc as plsc

N_NODES = 100000
N_EDGES = 1600000
D_IN = 128
D_HID = 64
D_OUT = 16
ALPHA = 0.1
K_PROP = 10

NPAD = 100096
SPAN = NPAD // 16
TRASH = N_NODES

STR = 128
KROW = 8
GITER = 50
RPW = KROW * GITER
EPAD = 32 * RPW * STR
CH = 1024
DEG_ITERS = EPAD // 32 // CH
NROW = NPAD * D_OUT // 128
R0 = 280
R1 = 520

_mesh = plsc.VectorSubcoreMesh(core_axis_name="c", subcore_axis_name="s")


def _mlp_body(x_ref, w1_ref, b1_ref, w2_ref, b2_ref, o_ref):
    h = jnp.dot(x_ref[...], w1_ref[...], preferred_element_type=jnp.float32)
    h = jnp.maximum(h + b1_ref[...], 0.0)
    o_ref[...] = jnp.dot(h, w2_ref[...], preferred_element_type=jnp.float32) + b2_ref[...]


def _mlp(x, w1, b1, w2, b2):
    return pl.pallas_call(
        _mlp_body,
        out_shape=jax.ShapeDtypeStruct((N_NODES, D_OUT), jnp.float32),
        grid=(25,),
        in_specs=[
            pl.BlockSpec((4000, D_IN), lambda i: (i, 0)),
            pl.BlockSpec((D_IN, D_HID), lambda i: (0, 0)),
            pl.BlockSpec((1, D_HID), lambda i: (0, 0)),
            pl.BlockSpec((D_HID, D_OUT), lambda i: (0, 0)),
            pl.BlockSpec((1, D_OUT), lambda i: (0, 0)),
        ],
        out_specs=pl.BlockSpec((4000, D_OUT), lambda i: (i, 0)),
    )(x, w1, b1, w2, b2)


@functools.partial(
    pl.kernel,
    out_type=(
        jax.ShapeDtypeStruct((32, NPAD), jnp.float32),
        jax.ShapeDtypeStruct((32, NPAD), jnp.float32),
    ),
    mesh=_mesh,
    scratch_types=[
        pltpu.VMEM((NPAD,), jnp.float32),
        pltpu.VMEM((CH,), jnp.int32),
    ],
    compiler_params=pltpu.CompilerParams(needs_layout_passes=False, use_tc_tiling_on_sc=False),
)
def _deg_kernel(src_hbm, dst_hbm, z1_hbm, dout_hbm, din_hbm, deg_l, idx_v):
    c = lax.axis_index("c")
    s = lax.axis_index("s")
    wid = s * 2 + c
    ebase = wid * (RPW * STR)
    ones16 = jnp.ones((16,), jnp.float32)

    def one_direction(e_hbm, o_hbm):
        pltpu.sync_copy(z1_hbm, deg_l)

        def outer(g, carry):
            pltpu.sync_copy(e_hbm.at[pl.ds(ebase + g * CH, CH)], idx_v)

            def inner(r, carry2):
                v = idx_v[pl.ds(r * 16, 16)]
                plsc.addupdate_scatter(deg_l, [v], ones16)
                return carry2

            return lax.fori_loop(0, CH // 16, inner, carry)

        lax.fori_loop(0, DEG_ITERS, outer, 0)
        pltpu.sync_copy(deg_l, o_hbm.at[wid])

    one_direction(src_hbm, dout_hbm)
    one_direction(dst_hbm, din_hbm)


def _norms_body(do_ref, di_ref, c1_ref, c2_ref, f1_ref, on_ref):
    od = jnp.clip(jnp.sum(do_ref[...], axis=0, keepdims=True), 1.0, None)
    idg = jnp.clip(jnp.sum(di_ref[...], axis=0, keepdims=True), 1.0, None)
    onv = lax.rsqrt(od)
    inv = lax.rsqrt(idg)
    c1_ref[...] = (1.0 - ALPHA) * onv * inv
    c2_ref[...] = ALPHA * onv
    f1_ref[...] = (1.0 - ALPHA) * inv
    on_ref[...] = onv


def _norms(dout, din):
    f32 = jnp.float32
    return pl.pallas_call(
        _norms_body,
        out_shape=(
            jax.ShapeDtypeStruct((1, NPAD), f32),
            jax.ShapeDtypeStruct((1, NPAD), f32),
            jax.ShapeDtypeStruct((1, NPAD), f32),
            jax.ShapeDtypeStruct((1, NPAD), f32),
        ),
        grid=(23,),
        in_specs=[
            pl.BlockSpec((32, 4352), lambda i: (0, i)),
            pl.BlockSpec((32, 4352), lambda i: (0, i)),
        ],
        out_specs=[
            pl.BlockSpec((1, 4352), lambda i: (0, i)),
            pl.BlockSpec((1, 4352), lambda i: (0, i)),
            pl.BlockSpec((1, 4352), lambda i: (0, i)),
            pl.BlockSpec((1, 4352), lambda i: (0, i)),
        ],
    )(dout, din)


def _scale_body(a_ref, x_ref, o_ref):
    o_ref[...] = a_ref[...] * x_ref[...]


def _scale(a_x, x):
    return pl.pallas_call(
        _scale_body,
        out_shape=jax.ShapeDtypeStruct((NROW, 128), jnp.float32),
        grid=(4,),
        in_specs=[
            pl.BlockSpec((3128, 128), lambda i: (i, 0)),
            pl.BlockSpec((3128, 128), lambda i: (i, 0)),
        ],
        out_specs=pl.BlockSpec((3128, 128), lambda i: (i, 0)),
    )(a_x, x)


def _blend_body(a_ref, b_ref, m_ref, h0_ref, o_ref):
    m = m_ref[0] + m_ref[1]
    o_ref[...] = a_ref[...] * m + b_ref[...] * h0_ref[...]


def _blend(a_x, b_x, parts_lin, h0_lin):
    return pl.pallas_call(
        _blend_body,
        out_shape=jax.ShapeDtypeStruct((NROW, 128), jnp.float32),
        grid=(4,),
        in_specs=[
            pl.BlockSpec((3128, 128), lambda i: (i, 0)),
            pl.BlockSpec((3128, 128), lambda i: (i, 0)),
            pl.BlockSpec((2, 3128, 128), lambda i: (0, i, 0)),
            pl.BlockSpec((3128, 128), lambda i: (i, 0)),
        ],
        out_specs=pl.BlockSpec((3128, 128), lambda i: (i, 0)),
    )(a_x, b_x, parts_lin, h0_lin)


@functools.partial(
    pl.kernel,
    out_type=jax.ShapeDtypeStruct((2, NPAD, D_OUT), jnp.float32),
    mesh=_mesh,
    scratch_types=[
        pltpu.VMEM_SHARED((NPAD, D_OUT), jnp.float32),
        pltpu.VMEM((KROW, STR), jnp.int32),
        pltpu.VMEM((KROW, STR), jnp.int32),
        pltpu.VMEM((KROW, STR, D_OUT), jnp.float32),
        pltpu.SemaphoreType.DMA,
        pltpu.SemaphoreType.DMA,
    ],
    compiler_params=pltpu.CompilerParams(needs_layout_passes=False, use_tc_tiling_on_sc=False),
)
def _scatter_kernel(hs_hbm, src_hbm, dst_hbm, z_hbm, out_hbm,
                    msg_sh, src_v, dst_v, rows_v, sem_g, sem_s):
    c = lax.axis_index("c")
    s = lax.axis_index("s")
    wid = s * 2 + c
    nbase = s * SPAN

    pltpu.sync_copy(z_hbm.at[pl.ds(nbase, SPAN)], msg_sh.at[pl.ds(nbase, SPAN)])
    plsc.subcore_barrier()

    ebase = jnp.where(c == 0, s * R0, 16 * R0 + s * R1)
    gmax = jnp.where(c == 0, R0 // KROW, R1 // KROW)

    def outer(g, carry):
        r0 = ebase + g * KROW
        pltpu.sync_copy(src_hbm.at[pl.ds(r0, KROW)], src_v)
        pltpu.sync_copy(dst_hbm.at[pl.ds(r0, KROW)], dst_v)
        gathers = [
            pltpu.async_copy(hs_hbm.at[src_v.at[j]], rows_v.at[j], sem_g)
            for j in range(KROW)
        ]
        for cp in gathers:
            cp.wait()
        scatters = [
            pltpu.async_copy(rows_v.at[j], msg_sh.at[dst_v.at[j]], sem_s, add=True)
            for j in range(KROW)
        ]
        for cp in scatters:
            cp.wait()
        return carry

    lax.fori_loop(0, gmax, outer, 0)
    plsc.subcore_barrier()
    pltpu.sync_copy(msg_sh.at[pl.ds(nbase, SPAN)], out_hbm.at[c, pl.ds(nbase, SPAN)])


def kernel(features, edge_index, W1, b1, W2, b2):
    f32 = jnp.float32
    src = edge_index[0]
    dst = edge_index[1]
    pad = EPAD - N_EDGES
    srcp = jnp.concatenate([src, jnp.full((pad,), TRASH, jnp.int32)])
    dstp = jnp.concatenate([dst, jnp.full((pad,), TRASH, jnp.int32)])
    src2 = srcp.reshape(EPAD // STR, STR)
    dst2 = dstp.reshape(EPAD // STR, STR)
    z1 = jnp.zeros((NPAD,), f32)
    z2 = jnp.zeros((NPAD, D_OUT), f32)

    h0 = _mlp(features, W1, b1.reshape(1, D_HID), W2, b2.reshape(1, D_OUT))
    h0p = jnp.concatenate([h0, jnp.zeros((NPAD - N_NODES, D_OUT), f32)], axis=0)
    h0_lin = h0p.reshape(NROW, 128)

    dout, din = _deg_kernel(srcp, dstp, z1)
    c1r, c2r, f1r, onr = _norms(dout, din)

    def expand(row):
        return jnp.repeat(row.reshape(NPAD), D_OUT).reshape(NROW, 128)

    c1x = expand(c1r)
    c2x = expand(c2r)
    f1x = expand(f1r)
    onx = expand(onr)
    alpha_x = jnp.full((NROW, 128), ALPHA, f32)

    hs = _scale(onx, h0_lin)
    for t in range(K_PROP):
        parts = _scatter_kernel(hs.reshape(NPAD, D_OUT), src2, dst2, z2)
        parts_lin = parts.reshape(2, NROW, 128)
        if t < K_PROP - 1:
            hs = _blend(c1x, c2x, parts_lin, h0_lin)
        else:
            hs = _blend(f1x, alpha_x, parts_lin, h0_lin)
    return hs.reshape(NPAD, D_OUT)[:N_NODES]


# speedup vs baseline: 1.2397x; 1.2397x over previous
import functools

import jax
import jax.numpy as jnp
from jax import lax
from jax.experimental import pallas as pl
from jax.experimental.pallas import tpu as pltpu
from jax.experimental.pallas import tpu_sc as plsc

N_NODES = 100000
N_EDGES = 1600000
D_IN = 128
D_HID = 64
D_OUT = 16
ALPHA = 0.1
K_PROP = 10

NPAD = 100096
SPAN = NPAD // 16
TRASH = N_NODES

STR = 128
KROW = 8
GITER = 50
RPW = KROW * GITER
EPAD = 32 * RPW * STR
CH = 1024
DEG_ITERS = EPAD // 32 // CH
NROW = NPAD * D_OUT // 128
R0 = 520
R1 = 280

_mesh = plsc.VectorSubcoreMesh(core_axis_name="c", subcore_axis_name="s")


def _mlp_body(x_ref, w1_ref, b1_ref, w2_ref, b2_ref, o_ref):
    h = jnp.dot(x_ref[...], w1_ref[...], preferred_element_type=jnp.float32)
    h = jnp.maximum(h + b1_ref[...], 0.0)
    o_ref[...] = jnp.dot(h, w2_ref[...], preferred_element_type=jnp.float32) + b2_ref[...]


def _mlp(x, w1, b1, w2, b2):
    return pl.pallas_call(
        _mlp_body,
        out_shape=jax.ShapeDtypeStruct((N_NODES, D_OUT), jnp.float32),
        grid=(25,),
        in_specs=[
            pl.BlockSpec((4000, D_IN), lambda i: (i, 0)),
            pl.BlockSpec((D_IN, D_HID), lambda i: (0, 0)),
            pl.BlockSpec((1, D_HID), lambda i: (0, 0)),
            pl.BlockSpec((D_HID, D_OUT), lambda i: (0, 0)),
            pl.BlockSpec((1, D_OUT), lambda i: (0, 0)),
        ],
        out_specs=pl.BlockSpec((4000, D_OUT), lambda i: (i, 0)),
    )(x, w1, b1, w2, b2)


@functools.partial(
    pl.kernel,
    out_type=(
        jax.ShapeDtypeStruct((32, NPAD), jnp.float32),
        jax.ShapeDtypeStruct((32, NPAD), jnp.float32),
    ),
    mesh=_mesh,
    scratch_types=[
        pltpu.VMEM((NPAD,), jnp.float32),
        pltpu.VMEM((CH,), jnp.int32),
    ],
    compiler_params=pltpu.CompilerParams(needs_layout_passes=False, use_tc_tiling_on_sc=False),
)
def _deg_kernel(src_hbm, dst_hbm, z1_hbm, dout_hbm, din_hbm, deg_l, idx_v):
    c = lax.axis_index("c")
    s = lax.axis_index("s")
    wid = s * 2 + c
    ebase = wid * (RPW * STR)
    ones16 = jnp.ones((16,), jnp.float32)

    def one_direction(e_hbm, o_hbm):
        pltpu.sync_copy(z1_hbm, deg_l)

        def outer(g, carry):
            pltpu.sync_copy(e_hbm.at[pl.ds(ebase + g * CH, CH)], idx_v)

            def inner(r, carry2):
                v = idx_v[pl.ds(r * 16, 16)]
                plsc.addupdate_scatter(deg_l, [v], ones16)
                return carry2

            return lax.fori_loop(0, CH // 16, inner, carry)

        lax.fori_loop(0, DEG_ITERS, outer, 0)
        pltpu.sync_copy(deg_l, o_hbm.at[wid])

    one_direction(src_hbm, dout_hbm)
    one_direction(dst_hbm, din_hbm)


def _norms_body(do_ref, di_ref, c1_ref, c2_ref, f1_ref, on_ref):
    od = jnp.clip(jnp.sum(do_ref[...], axis=0, keepdims=True), 1.0, None)
    idg = jnp.clip(jnp.sum(di_ref[...], axis=0, keepdims=True), 1.0, None)
    onv = lax.rsqrt(od)
    inv = lax.rsqrt(idg)
    c1_ref[...] = (1.0 - ALPHA) * onv * inv
    c2_ref[...] = ALPHA * onv
    f1_ref[...] = (1.0 - ALPHA) * inv
    on_ref[...] = onv


def _norms(dout, din):
    f32 = jnp.float32
    return pl.pallas_call(
        _norms_body,
        out_shape=(
            jax.ShapeDtypeStruct((1, NPAD), f32),
            jax.ShapeDtypeStruct((1, NPAD), f32),
            jax.ShapeDtypeStruct((1, NPAD), f32),
            jax.ShapeDtypeStruct((1, NPAD), f32),
        ),
        grid=(23,),
        in_specs=[
            pl.BlockSpec((32, 4352), lambda i: (0, i)),
            pl.BlockSpec((32, 4352), lambda i: (0, i)),
        ],
        out_specs=[
            pl.BlockSpec((1, 4352), lambda i: (0, i)),
            pl.BlockSpec((1, 4352), lambda i: (0, i)),
            pl.BlockSpec((1, 4352), lambda i: (0, i)),
            pl.BlockSpec((1, 4352), lambda i: (0, i)),
        ],
    )(dout, din)


def _scale_body(a_ref, x_ref, o_ref):
    o_ref[...] = a_ref[...] * x_ref[...]


def _scale(a_x, x):
    return pl.pallas_call(
        _scale_body,
        out_shape=jax.ShapeDtypeStruct((NROW, 128), jnp.float32),
        grid=(4,),
        in_specs=[
            pl.BlockSpec((3128, 128), lambda i: (i, 0)),
            pl.BlockSpec((3128, 128), lambda i: (i, 0)),
        ],
        out_specs=pl.BlockSpec((3128, 128), lambda i: (i, 0)),
    )(a_x, x)


def _blend_body(a_ref, b_ref, m_ref, h0_ref, o_ref):
    m = m_ref[0] + m_ref[1]
    o_ref[...] = a_ref[...] * m + b_ref[...] * h0_ref[...]


def _blend(a_x, b_x, parts_lin, h0_lin):
    return pl.pallas_call(
        _blend_body,
        out_shape=jax.ShapeDtypeStruct((NROW, 128), jnp.float32),
        grid=(4,),
        in_specs=[
            pl.BlockSpec((3128, 128), lambda i: (i, 0)),
            pl.BlockSpec((3128, 128), lambda i: (i, 0)),
            pl.BlockSpec((2, 3128, 128), lambda i: (0, i, 0)),
            pl.BlockSpec((3128, 128), lambda i: (i, 0)),
        ],
        out_specs=pl.BlockSpec((3128, 128), lambda i: (i, 0)),
    )(a_x, b_x, parts_lin, h0_lin)


@functools.partial(
    pl.kernel,
    out_type=jax.ShapeDtypeStruct((2, NPAD, D_OUT), jnp.float32),
    mesh=_mesh,
    scratch_types=[
        pltpu.VMEM_SHARED((NPAD, D_OUT), jnp.float32),
        pltpu.VMEM((KROW, STR), jnp.int32),
        pltpu.VMEM((KROW, STR), jnp.int32),
        pltpu.VMEM((KROW, STR, D_OUT), jnp.float32),
        pltpu.SemaphoreType.DMA,
        pltpu.SemaphoreType.DMA,
    ],
    compiler_params=pltpu.CompilerParams(needs_layout_passes=False, use_tc_tiling_on_sc=False),
)
def _scatter_kernel(hs_hbm, src_hbm, dst_hbm, z_hbm, out_hbm,
                    msg_sh, src_v, dst_v, rows_v, sem_g, sem_s):
    c = lax.axis_index("c")
    s = lax.axis_index("s")
    wid = s * 2 + c
    nbase = s * SPAN

    pltpu.sync_copy(z_hbm.at[pl.ds(nbase, SPAN)], msg_sh.at[pl.ds(nbase, SPAN)])
    plsc.subcore_barrier()

    ebase = jnp.where(c == 0, s * R0, 16 * R0 + s * R1)
    gmax = jnp.where(c == 0, R0 // KROW, R1 // KROW)

    def outer(g, carry):
        r0 = ebase + g * KROW
        pltpu.sync_copy(src_hbm.at[pl.ds(r0, KROW)], src_v)
        pltpu.sync_copy(dst_hbm.at[pl.ds(r0, KROW)], dst_v)
        gathers = [
            pltpu.async_copy(hs_hbm.at[src_v.at[j]], rows_v.at[j], sem_g)
            for j in range(KROW)
        ]
        for cp in gathers:
            cp.wait()
        scatters = [
            pltpu.async_copy(rows_v.at[j], msg_sh.at[dst_v.at[j]], sem_s, add=True)
            for j in range(KROW)
        ]
        for cp in scatters:
            cp.wait()
        return carry

    lax.fori_loop(0, gmax, outer, 0)
    plsc.subcore_barrier()
    pltpu.sync_copy(msg_sh.at[pl.ds(nbase, SPAN)], out_hbm.at[c, pl.ds(nbase, SPAN)])


def kernel(features, edge_index, W1, b1, W2, b2):
    f32 = jnp.float32
    src = edge_index[0]
    dst = edge_index[1]
    pad = EPAD - N_EDGES
    srcp = jnp.concatenate([src, jnp.full((pad,), TRASH, jnp.int32)])
    dstp = jnp.concatenate([dst, jnp.full((pad,), TRASH, jnp.int32)])
    src2 = srcp.reshape(EPAD // STR, STR)
    dst2 = dstp.reshape(EPAD // STR, STR)
    z1 = jnp.zeros((NPAD,), f32)
    z2 = jnp.zeros((NPAD, D_OUT), f32)

    h0 = _mlp(features, W1, b1.reshape(1, D_HID), W2, b2.reshape(1, D_OUT))
    h0p = jnp.concatenate([h0, jnp.zeros((NPAD - N_NODES, D_OUT), f32)], axis=0)
    h0_lin = h0p.reshape(NROW, 128)

    dout, din = _deg_kernel(srcp, dstp, z1)
    c1r, c2r, f1r, onr = _norms(dout, din)

    def expand(row):
        return jnp.repeat(row.reshape(NPAD), D_OUT).reshape(NROW, 128)

    c1x = expand(c1r)
    c2x = expand(c2r)
    f1x = expand(f1r)
    onx = expand(onr)
    alpha_x = jnp.full((NROW, 128), ALPHA, f32)

    hs = _scale(onx, h0_lin)
    for t in range(K_PROP):
        parts = _scatter_kernel(hs.reshape(NPAD, D_OUT), src2, dst2, z2)
        parts_lin = parts.reshape(2, NROW, 128)
        if t < K_PROP - 1:
            hs = _blend(c1x, c2x, parts_lin, h0_lin)
        else:
            hs = _blend(f1x, alpha_x, parts_lin, h0_lin)
    return hs.reshape(NPAD, D_OUT)[:N_NODES]


# speedup vs baseline: 1.3117x; 1.0581x over previous
import functools

import jax
import jax.numpy as jnp
from jax import lax
from jax.experimental import pallas as pl
from jax.experimental.pallas import tpu as pltpu
from jax.experimental.pallas import tpu_sc as plsc

N_NODES = 100000
N_EDGES = 1600000
D_IN = 128
D_HID = 64
D_OUT = 16
ALPHA = 0.1
K_PROP = 10

NPAD = 100096
SPAN = NPAD // 16
TRASH = N_NODES

STR = 128
KROW = 8
GITER = 50
RPW = KROW * GITER
EPAD = 32 * RPW * STR
CH = 1024
DEG_ITERS = EPAD // 32 // CH
NROW = NPAD * D_OUT // 128
R0 = 552
R1 = 248
DI0 = 57
DI1 = 43

_mesh = plsc.VectorSubcoreMesh(core_axis_name="c", subcore_axis_name="s")


def _mlp_body(x_ref, w1_ref, b1_ref, w2_ref, b2_ref, o_ref):
    h = jnp.dot(x_ref[...], w1_ref[...], preferred_element_type=jnp.float32)
    h = jnp.maximum(h + b1_ref[...], 0.0)
    o_ref[...] = jnp.dot(h, w2_ref[...], preferred_element_type=jnp.float32) + b2_ref[...]


def _mlp(x, w1, b1, w2, b2):
    return pl.pallas_call(
        _mlp_body,
        out_shape=jax.ShapeDtypeStruct((N_NODES, D_OUT), jnp.float32),
        grid=(25,),
        in_specs=[
            pl.BlockSpec((4000, D_IN), lambda i: (i, 0)),
            pl.BlockSpec((D_IN, D_HID), lambda i: (0, 0)),
            pl.BlockSpec((1, D_HID), lambda i: (0, 0)),
            pl.BlockSpec((D_HID, D_OUT), lambda i: (0, 0)),
            pl.BlockSpec((1, D_OUT), lambda i: (0, 0)),
        ],
        out_specs=pl.BlockSpec((4000, D_OUT), lambda i: (i, 0)),
    )(x, w1, b1, w2, b2)


@functools.partial(
    pl.kernel,
    out_type=(
        jax.ShapeDtypeStruct((32, NPAD), jnp.float32),
        jax.ShapeDtypeStruct((32, NPAD), jnp.float32),
    ),
    mesh=_mesh,
    scratch_types=[
        pltpu.VMEM((NPAD,), jnp.float32),
        pltpu.VMEM((CH,), jnp.int32),
    ],
    compiler_params=pltpu.CompilerParams(needs_layout_passes=False, use_tc_tiling_on_sc=False),
)
def _deg_kernel(src_hbm, dst_hbm, z1_hbm, dout_hbm, din_hbm, deg_l, idx_v):
    c = lax.axis_index("c")
    s = lax.axis_index("s")
    wid = s * 2 + c
    ebase = jnp.where(c == 0, s * (DI0 * CH), 16 * DI0 * CH + s * (DI1 * CH))
    diters = jnp.where(c == 0, DI0, DI1)
    ones16 = jnp.ones((16,), jnp.float32)

    def one_direction(e_hbm, o_hbm):
        pltpu.sync_copy(z1_hbm, deg_l)

        def outer(g, carry):
            pltpu.sync_copy(e_hbm.at[pl.ds(ebase + g * CH, CH)], idx_v)

            def inner(r, carry2):
                v = idx_v[pl.ds(r * 16, 16)]
                plsc.addupdate_scatter(deg_l, [v], ones16)
                return carry2

            return lax.fori_loop(0, CH // 16, inner, carry)

        lax.fori_loop(0, diters, outer, 0)
        pltpu.sync_copy(deg_l, o_hbm.at[wid])

    one_direction(src_hbm, dout_hbm)
    one_direction(dst_hbm, din_hbm)


def _norms_body(do_ref, di_ref, c1_ref, c2_ref, f1_ref, on_ref):
    od = jnp.clip(jnp.sum(do_ref[...], axis=0, keepdims=True), 1.0, None)
    idg = jnp.clip(jnp.sum(di_ref[...], axis=0, keepdims=True), 1.0, None)
    onv = lax.rsqrt(od)
    inv = lax.rsqrt(idg)
    c1_ref[...] = (1.0 - ALPHA) * onv * inv
    c2_ref[...] = ALPHA * onv
    f1_ref[...] = (1.0 - ALPHA) * inv
    on_ref[...] = onv


def _norms(dout, din):
    f32 = jnp.float32
    return pl.pallas_call(
        _norms_body,
        out_shape=(
            jax.ShapeDtypeStruct((1, NPAD), f32),
            jax.ShapeDtypeStruct((1, NPAD), f32),
            jax.ShapeDtypeStruct((1, NPAD), f32),
            jax.ShapeDtypeStruct((1, NPAD), f32),
        ),
        grid=(23,),
        in_specs=[
            pl.BlockSpec((32, 4352), lambda i: (0, i)),
            pl.BlockSpec((32, 4352), lambda i: (0, i)),
        ],
        out_specs=[
            pl.BlockSpec((1, 4352), lambda i: (0, i)),
            pl.BlockSpec((1, 4352), lambda i: (0, i)),
            pl.BlockSpec((1, 4352), lambda i: (0, i)),
            pl.BlockSpec((1, 4352), lambda i: (0, i)),
        ],
    )(dout, din)


def _scale_body(a_ref, x_ref, o_ref):
    o_ref[...] = a_ref[...] * x_ref[...]


def _scale(a_x, x):
    return pl.pallas_call(
        _scale_body,
        out_shape=jax.ShapeDtypeStruct((NROW, 128), jnp.float32),
        grid=(4,),
        in_specs=[
            pl.BlockSpec((3128, 128), lambda i: (i, 0)),
            pl.BlockSpec((3128, 128), lambda i: (i, 0)),
        ],
        out_specs=pl.BlockSpec((3128, 128), lambda i: (i, 0)),
    )(a_x, x)


def _blend_body(a_ref, b_ref, m_ref, h0_ref, o_ref):
    m = m_ref[0] + m_ref[1]
    o_ref[...] = a_ref[...] * m + b_ref[...] * h0_ref[...]


def _blend(a_x, b_x, parts_lin, h0_lin):
    return pl.pallas_call(
        _blend_body,
        out_shape=jax.ShapeDtypeStruct((NROW, 128), jnp.float32),
        grid=(4,),
        in_specs=[
            pl.BlockSpec((3128, 128), lambda i: (i, 0)),
            pl.BlockSpec((3128, 128), lambda i: (i, 0)),
            pl.BlockSpec((2, 3128, 128), lambda i: (0, i, 0)),
            pl.BlockSpec((3128, 128), lambda i: (i, 0)),
        ],
        out_specs=pl.BlockSpec((3128, 128), lambda i: (i, 0)),
    )(a_x, b_x, parts_lin, h0_lin)


@functools.partial(
    pl.kernel,
    out_type=jax.ShapeDtypeStruct((2, NPAD, D_OUT), jnp.float32),
    mesh=_mesh,
    scratch_types=[
        pltpu.VMEM_SHARED((NPAD, D_OUT), jnp.float32),
        pltpu.VMEM((KROW, STR), jnp.int32),
        pltpu.VMEM((KROW, STR), jnp.int32),
        pltpu.VMEM((KROW, STR, D_OUT), jnp.float32),
        pltpu.SemaphoreType.DMA,
        pltpu.SemaphoreType.DMA,
    ],
    compiler_params=pltpu.CompilerParams(needs_layout_passes=False, use_tc_tiling_on_sc=False),
)
def _scatter_kernel(hs_hbm, src_hbm, dst_hbm, z_hbm, out_hbm,
                    msg_sh, src_v, dst_v, rows_v, sem_g, sem_s):
    c = lax.axis_index("c")
    s = lax.axis_index("s")
    wid = s * 2 + c
    nbase = s * SPAN

    pltpu.sync_copy(z_hbm.at[pl.ds(nbase, SPAN)], msg_sh.at[pl.ds(nbase, SPAN)])
    plsc.subcore_barrier()

    ebase = jnp.where(c == 0, s * R0, 16 * R0 + s * R1)
    gmax = jnp.where(c == 0, R0 // KROW, R1 // KROW)

    def outer(g, carry):
        r0 = ebase + g * KROW
        pltpu.sync_copy(src_hbm.at[pl.ds(r0, KROW)], src_v)
        pltpu.sync_copy(dst_hbm.at[pl.ds(r0, KROW)], dst_v)
        gathers = [
            pltpu.async_copy(hs_hbm.at[src_v.at[j]], rows_v.at[j], sem_g)
            for j in range(KROW)
        ]
        for cp in gathers:
            cp.wait()
        scatters = [
            pltpu.async_copy(rows_v.at[j], msg_sh.at[dst_v.at[j]], sem_s, add=True)
            for j in range(KROW)
        ]
        for cp in scatters:
            cp.wait()
        return carry

    lax.fori_loop(0, gmax, outer, 0)
    plsc.subcore_barrier()
    pltpu.sync_copy(msg_sh.at[pl.ds(nbase, SPAN)], out_hbm.at[c, pl.ds(nbase, SPAN)])


def kernel(features, edge_index, W1, b1, W2, b2):
    f32 = jnp.float32
    src = edge_index[0]
    dst = edge_index[1]
    pad = EPAD - N_EDGES
    srcp = jnp.concatenate([src, jnp.full((pad,), TRASH, jnp.int32)])
    dstp = jnp.concatenate([dst, jnp.full((pad,), TRASH, jnp.int32)])
    src2 = srcp.reshape(EPAD // STR, STR)
    dst2 = dstp.reshape(EPAD // STR, STR)
    z1 = jnp.zeros((NPAD,), f32)
    z2 = jnp.zeros((NPAD, D_OUT), f32)

    h0 = _mlp(features, W1, b1.reshape(1, D_HID), W2, b2.reshape(1, D_OUT))
    h0p = jnp.concatenate([h0, jnp.zeros((NPAD - N_NODES, D_OUT), f32)], axis=0)
    h0_lin = h0p.reshape(NROW, 128)

    dout, din = _deg_kernel(srcp, dstp, z1)
    c1r, c2r, f1r, onr = _norms(dout, din)

    def expand(row):
        return jnp.repeat(row.reshape(NPAD), D_OUT).reshape(NROW, 128)

    c1x = expand(c1r)
    c2x = expand(c2r)
    f1x = expand(f1r)
    onx = expand(onr)
    alpha_x = jnp.full((NROW, 128), ALPHA, f32)

    hs = _scale(onx, h0_lin)
    for t in range(K_PROP):
        parts = _scatter_kernel(hs.reshape(NPAD, D_OUT), src2, dst2, z2)
        parts_lin = parts.reshape(2, NROW, 128)
        if t < K_PROP - 1:
            hs = _blend(c1x, c2x, parts_lin, h0_lin)
        else:
            hs = _blend(f1x, alpha_x, parts_lin, h0_lin)
    return hs.reshape(NPAD, D_OUT)[:N_NODES]


# speedup vs baseline: 1.3402x; 1.0217x over previous
import functools

import jax
import jax.numpy as jnp
from jax import lax
from jax.experimental import pallas as pl
from jax.experimental.pallas import tpu as pltpu
from jax.experimental.pallas import tpu_sc as plsc

N_NODES = 100000
N_EDGES = 1600000
D_IN = 128
D_HID = 64
D_OUT = 16
ALPHA = 0.1
K_PROP = 10

NPAD = 100096
SPAN = NPAD // 16
TRASH = N_NODES

STR = 128
KROW = 8
GITER = 50
RPW = KROW * GITER
EPAD = 32 * RPW * STR
CH = 1024
DEG_ITERS = EPAD // 32 // CH
NROW = NPAD * D_OUT // 128
R0 = 568
R1 = 232
DI0 = 57
DI1 = 43

_mesh = plsc.VectorSubcoreMesh(core_axis_name="c", subcore_axis_name="s")


def _mlp_body(x_ref, w1_ref, b1_ref, w2_ref, b2_ref, o_ref):
    h = jnp.dot(x_ref[...], w1_ref[...], preferred_element_type=jnp.float32)
    h = jnp.maximum(h + b1_ref[...], 0.0)
    o_ref[...] = jnp.dot(h, w2_ref[...], preferred_element_type=jnp.float32) + b2_ref[...]


def _mlp(x, w1, b1, w2, b2):
    return pl.pallas_call(
        _mlp_body,
        out_shape=jax.ShapeDtypeStruct((N_NODES, D_OUT), jnp.float32),
        grid=(25,),
        in_specs=[
            pl.BlockSpec((4000, D_IN), lambda i: (i, 0)),
            pl.BlockSpec((D_IN, D_HID), lambda i: (0, 0)),
            pl.BlockSpec((1, D_HID), lambda i: (0, 0)),
            pl.BlockSpec((D_HID, D_OUT), lambda i: (0, 0)),
            pl.BlockSpec((1, D_OUT), lambda i: (0, 0)),
        ],
        out_specs=pl.BlockSpec((4000, D_OUT), lambda i: (i, 0)),
    )(x, w1, b1, w2, b2)


@functools.partial(
    pl.kernel,
    out_type=(
        jax.ShapeDtypeStruct((32, NPAD), jnp.float32),
        jax.ShapeDtypeStruct((32, NPAD), jnp.float32),
    ),
    mesh=_mesh,
    scratch_types=[
        pltpu.VMEM((NPAD,), jnp.float32),
        pltpu.VMEM((CH,), jnp.int32),
    ],
    compiler_params=pltpu.CompilerParams(needs_layout_passes=False, use_tc_tiling_on_sc=False),
)
def _deg_kernel(src_hbm, dst_hbm, z1_hbm, dout_hbm, din_hbm, deg_l, idx_v):
    c = lax.axis_index("c")
    s = lax.axis_index("s")
    wid = s * 2 + c
    ebase = jnp.where(c == 0, s * (DI0 * CH), 16 * DI0 * CH + s * (DI1 * CH))
    diters = jnp.where(c == 0, DI0, DI1)
    ones16 = jnp.ones((16,), jnp.float32)

    def one_direction(e_hbm, o_hbm):
        pltpu.sync_copy(z1_hbm, deg_l)

        def outer(g, carry):
            pltpu.sync_copy(e_hbm.at[pl.ds(ebase + g * CH, CH)], idx_v)

            def inner(r, carry2):
                v = idx_v[pl.ds(r * 16, 16)]
                plsc.addupdate_scatter(deg_l, [v], ones16)
                return carry2

            return lax.fori_loop(0, CH // 16, inner, carry)

        lax.fori_loop(0, diters, outer, 0)
        pltpu.sync_copy(deg_l, o_hbm.at[wid])

    one_direction(src_hbm, dout_hbm)
    one_direction(dst_hbm, din_hbm)


def _norms_body(do_ref, di_ref, c1_ref, c2_ref, f1_ref, on_ref):
    od = jnp.clip(jnp.sum(do_ref[...], axis=0, keepdims=True), 1.0, None)
    idg = jnp.clip(jnp.sum(di_ref[...], axis=0, keepdims=True), 1.0, None)
    onv = lax.rsqrt(od)
    inv = lax.rsqrt(idg)
    c1_ref[...] = (1.0 - ALPHA) * onv * inv
    c2_ref[...] = ALPHA * onv
    f1_ref[...] = (1.0 - ALPHA) * inv
    on_ref[...] = onv


def _norms(dout, din):
    f32 = jnp.float32
    return pl.pallas_call(
        _norms_body,
        out_shape=(
            jax.ShapeDtypeStruct((1, NPAD), f32),
            jax.ShapeDtypeStruct((1, NPAD), f32),
            jax.ShapeDtypeStruct((1, NPAD), f32),
            jax.ShapeDtypeStruct((1, NPAD), f32),
        ),
        grid=(23,),
        in_specs=[
            pl.BlockSpec((32, 4352), lambda i: (0, i)),
            pl.BlockSpec((32, 4352), lambda i: (0, i)),
        ],
        out_specs=[
            pl.BlockSpec((1, 4352), lambda i: (0, i)),
            pl.BlockSpec((1, 4352), lambda i: (0, i)),
            pl.BlockSpec((1, 4352), lambda i: (0, i)),
            pl.BlockSpec((1, 4352), lambda i: (0, i)),
        ],
    )(dout, din)


def _scale_body(a_ref, x_ref, o_ref):
    o_ref[...] = a_ref[...] * x_ref[...]


def _scale(a_x, x):
    return pl.pallas_call(
        _scale_body,
        out_shape=jax.ShapeDtypeStruct((NROW, 128), jnp.float32),
        grid=(4,),
        in_specs=[
            pl.BlockSpec((3128, 128), lambda i: (i, 0)),
            pl.BlockSpec((3128, 128), lambda i: (i, 0)),
        ],
        out_specs=pl.BlockSpec((3128, 128), lambda i: (i, 0)),
    )(a_x, x)


def _blend_body(a_ref, b_ref, m_ref, h0_ref, o_ref):
    m = m_ref[0] + m_ref[1]
    o_ref[...] = a_ref[...] * m + b_ref[...] * h0_ref[...]


def _blend(a_x, b_x, parts_lin, h0_lin):
    return pl.pallas_call(
        _blend_body,
        out_shape=jax.ShapeDtypeStruct((NROW, 128), jnp.float32),
        grid=(4,),
        in_specs=[
            pl.BlockSpec((3128, 128), lambda i: (i, 0)),
            pl.BlockSpec((3128, 128), lambda i: (i, 0)),
            pl.BlockSpec((2, 3128, 128), lambda i: (0, i, 0)),
            pl.BlockSpec((3128, 128), lambda i: (i, 0)),
        ],
        out_specs=pl.BlockSpec((3128, 128), lambda i: (i, 0)),
    )(a_x, b_x, parts_lin, h0_lin)


@functools.partial(
    pl.kernel,
    out_type=jax.ShapeDtypeStruct((2, NPAD, D_OUT), jnp.float32),
    mesh=_mesh,
    scratch_types=[
        pltpu.VMEM_SHARED((NPAD, D_OUT), jnp.float32),
        pltpu.VMEM((KROW, STR), jnp.int32),
        pltpu.VMEM((KROW, STR), jnp.int32),
        pltpu.VMEM((KROW, STR, D_OUT), jnp.float32),
        pltpu.SemaphoreType.DMA,
        pltpu.SemaphoreType.DMA,
    ],
    compiler_params=pltpu.CompilerParams(needs_layout_passes=False, use_tc_tiling_on_sc=False),
)
def _scatter_kernel(hs_hbm, src_hbm, dst_hbm, z_hbm, out_hbm,
                    msg_sh, src_v, dst_v, rows_v, sem_g, sem_s):
    c = lax.axis_index("c")
    s = lax.axis_index("s")
    wid = s * 2 + c
    nbase = s * SPAN

    pltpu.sync_copy(z_hbm.at[pl.ds(nbase, SPAN)], msg_sh.at[pl.ds(nbase, SPAN)])
    plsc.subcore_barrier()

    ebase = jnp.where(c == 0, s * R0, 16 * R0 + s * R1)
    gmax = jnp.where(c == 0, R0 // KROW, R1 // KROW)

    def outer(g, carry):
        r0 = ebase + g * KROW
        pltpu.sync_copy(src_hbm.at[pl.ds(r0, KROW)], src_v)
        pltpu.sync_copy(dst_hbm.at[pl.ds(r0, KROW)], dst_v)
        gathers = [
            pltpu.async_copy(hs_hbm.at[src_v.at[j]], rows_v.at[j], sem_g)
            for j in range(KROW)
        ]
        for cp in gathers:
            cp.wait()
        scatters = [
            pltpu.async_copy(rows_v.at[j], msg_sh.at[dst_v.at[j]], sem_s, add=True)
            for j in range(KROW)
        ]
        for cp in scatters:
            cp.wait()
        return carry

    lax.fori_loop(0, gmax, outer, 0)
    plsc.subcore_barrier()
    pltpu.sync_copy(msg_sh.at[pl.ds(nbase, SPAN)], out_hbm.at[c, pl.ds(nbase, SPAN)])


def kernel(features, edge_index, W1, b1, W2, b2):
    f32 = jnp.float32
    src = edge_index[0]
    dst = edge_index[1]
    pad = EPAD - N_EDGES
    srcp = jnp.concatenate([src, jnp.full((pad,), TRASH, jnp.int32)])
    dstp = jnp.concatenate([dst, jnp.full((pad,), TRASH, jnp.int32)])
    src2 = srcp.reshape(EPAD // STR, STR)
    dst2 = dstp.reshape(EPAD // STR, STR)
    z1 = jnp.zeros((NPAD,), f32)
    z2 = jnp.zeros((NPAD, D_OUT), f32)

    h0 = _mlp(features, W1, b1.reshape(1, D_HID), W2, b2.reshape(1, D_OUT))
    h0p = jnp.concatenate([h0, jnp.zeros((NPAD - N_NODES, D_OUT), f32)], axis=0)
    h0_lin = h0p.reshape(NROW, 128)

    dout, din = _deg_kernel(srcp, dstp, z1)
    c1r, c2r, f1r, onr = _norms(dout, din)

    def expand(row):
        return jnp.repeat(row.reshape(NPAD), D_OUT).reshape(NROW, 128)

    c1x = expand(c1r)
    c2x = expand(c2r)
    f1x = expand(f1r)
    onx = expand(onr)
    alpha_x = jnp.full((NROW, 128), ALPHA, f32)

    hs = _scale(onx, h0_lin)
    for t in range(K_PROP):
        parts = _scatter_kernel(hs.reshape(NPAD, D_OUT), src2, dst2, z2)
        parts_lin = parts.reshape(2, NROW, 128)
        if t < K_PROP - 1:
            hs = _blend(c1x, c2x, parts_lin, h0_lin)
        else:
            hs = _blend(f1x, alpha_x, parts_lin, h0_lin)
    return hs.reshape(NPAD, D_OUT)[:N_NODES]
